# baseline (speedup 1.0000x reference)
import jax
import jax.numpy as jnp
from jax import lax
from jax.experimental import pallas as pl
from jax.experimental.pallas import tpu as pltpu
from jax.experimental.pallas import tpu_sc as plsc

_R, _N = 128, 32768
_RSC = 64
_ROFF = _R - _RSC
_L = 16
_B = _N // 2
_C = _B // _L
_G = _C // _L
_T = _G // _L
_NC, _NS = 2, 16
_NW = _NC * _NS
_RPW = _RSC // _NW
_NB = _RPW * 2
_RT = _ROFF
_RTB = 64
_SUB = 256
_NSUB = 8
_BT = _SUB * _NSUB


def _sc_body(x_hbm, m_hbm, o_hbm, xv, mv, ov, sums, sg, go, off, sems):
    wid = lax.axis_index("s") * _NC + lax.axis_index("c")
    lane = lax.iota(jnp.int32, _L)

    def start_in(b):
        p = b % 2
        row = wid * _RPW + b // 2
        sl = pl.ds((b % 2) * _B, _B)
        return (
            pltpu.async_copy(x_hbm.at[_ROFF + row, sl], xv.at[p], sems.at[p]),
            pltpu.async_copy(m_hbm.at[row, sl], mv.at[p], sems.at[2 + p]),
        )

    def compute_block(p, base):
        pidx = jnp.full((_L,), p, jnp.int32)

        @plsc.parallel_loop(0, _C, unroll=8)
        def _(i):
            o = i * _L
            ov[p, pl.ds(o, _L)] = jnp.cumsum(
                xv[p, pl.ds(o, _L)] * mv[p, pl.ds(o, _L)]
            )

        @plsc.parallel_loop(0, _G, unroll=4)
        def _(g):
            idx = (g * _L + lane) * _L + (_L - 1)
            sums[pl.ds(g * _L, _L)] = plsc.load_gather(ov, [pidx, idx])

        @plsc.parallel_loop(0, _G, unroll=4)
        def _(g):
            sg[pl.ds(g * _L, _L)] = jnp.cumsum(sums[pl.ds(g * _L, _L)])

        def p3(t, carry):
            idx = (t * _L + lane) * _L + (_L - 1)
            gt = plsc.load_gather(sg, [idx])
            st = jnp.cumsum(gt)
            go[pl.ds(t * _L, _L)] = st - gt + carry
            return carry + jnp.sum(gt)
        total = lax.fori_loop(0, _T, p3, base)

        @plsc.parallel_loop(0, _T)
        def _(t):
            gov = go[pl.ds(t * _L, _L)]
            for j in range(_L):
                o = (t * _L + j) * _L
                off[pl.ds(o, _L)] = (
                    sg[pl.ds(o, _L)] - sums[pl.ds(o, _L)] + gov[j]
                )

        @plsc.parallel_loop(0, _G, unroll=4)
        def _(g):
            offv = off[pl.ds(g * _L, _L)]
            for j in range(_L):
                o = (g * _L + j) * _L
                ov[p, pl.ds(o, _L)] = ov[p, pl.ds(o, _L)] + offv[j]

        return total

    in_cps = {0: start_in(0), 1: start_in(1)}
    out_cps = {}
    base = jnp.float32(0.0)
    for b in range(_NB):
        p = b % 2
        if b >= 2:
            out_cps.pop(b - 2).wait()
        for cp in in_cps.pop(b):
            cp.wait()
        if b % 2 == 0:
            base = jnp.float32(0.0)
        base = compute_block(p, base)
        if b + 2 < _NB:
            in_cps[b + 2] = start_in(b + 2)
        row = wid * _RPW + b // 2
        out_cps[b] = pltpu.async_copy(
            ov.at[p], o_hbm.at[row, pl.ds((b % 2) * _B, _B)], sems.at[4 + p]
        )
    for cp in out_cps.values():
        cp.wait()


def _sc_call(x, maskf):
    f = pl.kernel(
        _sc_body,
        out_type=jax.ShapeDtypeStruct((_RSC, _N), jnp.float32),
        mesh=plsc.VectorSubcoreMesh(core_axis_name="c", subcore_axis_name="s"),
        scratch_types=[
            pltpu.VMEM((2, _B), jnp.float32),
            pltpu.VMEM((2, _B), jnp.float32),
            pltpu.VMEM((2, _B), jnp.float32),
            pltpu.VMEM((_C,), jnp.float32),
            pltpu.VMEM((_C,), jnp.float32),
            pltpu.VMEM((_G,), jnp.float32),
            pltpu.VMEM((_C,), jnp.float32),
            pltpu.SemaphoreType.DMA((6,)),
        ],
        compiler_params=pltpu.CompilerParams(needs_layout_passes=False),
    )
    return f(x, maskf)


def _tc_body(x_ref, m_ref, tri_ref, o_ref, carry_ref):
    i = pl.program_id(0)

    @pl.when(i == 0)
    def _():
        carry_ref[...] = jnp.zeros_like(carry_ref)

    masked = jnp.where(m_ref[...], x_ref[...], 0.0)
    run = carry_ref[:, :1]
    offs = []
    for s in range(_NSUB):
        offs.append(run)
        sub_tot = jnp.sum(
            masked[:, s * _SUB : (s + 1) * _SUB], axis=1, keepdims=True
        )
        run = run + sub_tot
    carry_ref[...] = jnp.broadcast_to(run, carry_ref.shape)
    tri = tri_ref[...]
    for s in range(_NSUB):
        intra = lax.dot_general(
            masked[:, s * _SUB : (s + 1) * _SUB],
            tri,
            (((1,), (0,)), ((), ())),
            preferred_element_type=jnp.float32,
        )
        o_ref[:, s * _SUB : (s + 1) * _SUB] = intra + offs[s]


def _tc_call(x, mask, tri):
    return pl.pallas_call(
        _tc_body,
        grid=(_N // _BT,),
        in_specs=[
            pl.BlockSpec((_RTB, _BT), lambda i: (0, i)),
            pl.BlockSpec((_RTB, _BT), lambda i: (0, i)),
            pl.BlockSpec((_SUB, _SUB), lambda i: (0, 0)),
        ],
        out_specs=pl.BlockSpec((_RTB, _BT), lambda i: (0, i)),
        out_shape=jax.ShapeDtypeStruct((_R, _N), jnp.float32),
        scratch_shapes=[pltpu.VMEM((_RTB, 128), jnp.float32)],
        compiler_params=pltpu.CompilerParams(
            dimension_semantics=("arbitrary",),
        ),
    )(x, mask, tri)


def kernel(x, mask):
    maskf = mask[_ROFF:].astype(jnp.float32)
    tri = jnp.triu(jnp.ones((_SUB, _SUB), jnp.float32))
    sc_out = _sc_call(x, maskf)
    tc_out = _tc_call(x, mask, tri)
    return lax.dynamic_update_slice(tc_out, sc_out, (_ROFF, 0))


# speedup vs baseline: 1.1975x; 1.1975x over previous
import jax
import jax.numpy as jnp
from jax import lax
from jax.experimental import pallas as pl
from jax.experimental.pallas import tpu as pltpu
from jax.experimental.pallas import tpu_sc as plsc

_R, _N = 128, 32768
_RSC = 32
_ROFF = _R - _RSC
_L = 16
_B = _N // 2
_C = _B // _L
_G = _C // _L
_T = _G // _L
_NC, _NS = 2, 16
_NW = _NC * _NS
_RPW = _RSC // _NW
_NB = _RPW * 2
_RT = _ROFF
_RTB = 96
_SUB = 256
_NSUB = 16
_BT = _SUB * _NSUB


def _sc_body(x_hbm, m_hbm, o_hbm, xv, mv, ov, sums, sg, go, off, sems):
    wid = lax.axis_index("s") * _NC + lax.axis_index("c")
    lane = lax.iota(jnp.int32, _L)

    def start_in(b):
        p = b % 2
        row = wid * _RPW + b // 2
        sl = pl.ds((b % 2) * _B, _B)
        return (
            pltpu.async_copy(x_hbm.at[_ROFF + row, sl], xv.at[p], sems.at[p]),
            pltpu.async_copy(m_hbm.at[row, sl], mv.at[p], sems.at[2 + p]),
        )

    def compute_block(p, base):
        pidx = jnp.full((_L,), p, jnp.int32)

        @plsc.parallel_loop(0, _C, unroll=8)
        def _(i):
            o = i * _L
            ov[p, pl.ds(o, _L)] = jnp.cumsum(
                xv[p, pl.ds(o, _L)] * mv[p, pl.ds(o, _L)]
            )

        @plsc.parallel_loop(0, _G, unroll=4)
        def _(g):
            idx = (g * _L + lane) * _L + (_L - 1)
            sums[pl.ds(g * _L, _L)] = plsc.load_gather(ov, [pidx, idx])

        @plsc.parallel_loop(0, _G, unroll=4)
        def _(g):
            sg[pl.ds(g * _L, _L)] = jnp.cumsum(sums[pl.ds(g * _L, _L)])

        def p3(t, carry):
            idx = (t * _L + lane) * _L + (_L - 1)
            gt = plsc.load_gather(sg, [idx])
            st = jnp.cumsum(gt)
            go[pl.ds(t * _L, _L)] = st - gt + carry
            return carry + jnp.sum(gt)
        total = lax.fori_loop(0, _T, p3, base)

        @plsc.parallel_loop(0, _T)
        def _(t):
            gov = go[pl.ds(t * _L, _L)]
            for j in range(_L):
                o = (t * _L + j) * _L
                off[pl.ds(o, _L)] = (
                    sg[pl.ds(o, _L)] - sums[pl.ds(o, _L)] + gov[j]
                )

        @plsc.parallel_loop(0, _G, unroll=4)
        def _(g):
            offv = off[pl.ds(g * _L, _L)]
            for j in range(_L):
                o = (g * _L + j) * _L
                ov[p, pl.ds(o, _L)] = ov[p, pl.ds(o, _L)] + offv[j]

        return total

    in_cps = {0: start_in(0), 1: start_in(1)}
    out_cps = {}
    base = jnp.float32(0.0)
    for b in range(_NB):
        p = b % 2
        if b >= 2:
            out_cps.pop(b - 2).wait()
        for cp in in_cps.pop(b):
            cp.wait()
        if b % 2 == 0:
            base = jnp.float32(0.0)
        base = compute_block(p, base)
        if b + 2 < _NB:
            in_cps[b + 2] = start_in(b + 2)
        row = wid * _RPW + b // 2
        out_cps[b] = pltpu.async_copy(
            ov.at[p], o_hbm.at[row, pl.ds((b % 2) * _B, _B)], sems.at[4 + p]
        )
    for cp in out_cps.values():
        cp.wait()


def _sc_call(x, maskf):
    f = pl.kernel(
        _sc_body,
        out_type=jax.ShapeDtypeStruct((_RSC, _N), jnp.float32),
        mesh=plsc.VectorSubcoreMesh(core_axis_name="c", subcore_axis_name="s"),
        scratch_types=[
            pltpu.VMEM((2, _B), jnp.float32),
            pltpu.VMEM((2, _B), jnp.float32),
            pltpu.VMEM((2, _B), jnp.float32),
            pltpu.VMEM((_C,), jnp.float32),
            pltpu.VMEM((_C,), jnp.float32),
            pltpu.VMEM((_G,), jnp.float32),
            pltpu.VMEM((_C,), jnp.float32),
            pltpu.SemaphoreType.DMA((6,)),
        ],
        compiler_params=pltpu.CompilerParams(needs_layout_passes=False),
    )
    return f(x, maskf)


def _tc_body(x_ref, m_ref, tri_ref, o_ref, carry_ref):
    i = pl.program_id(0)

    @pl.when(i == 0)
    def _():
        carry_ref[...] = jnp.zeros_like(carry_ref)

    masked = jnp.where(m_ref[...], x_ref[...], 0.0)
    run = carry_ref[:, :1]
    offs = []
    for s in range(_NSUB):
        offs.append(run)
        sub_tot = jnp.sum(
            masked[:, s * _SUB : (s + 1) * _SUB], axis=1, keepdims=True
        )
        run = run + sub_tot
    carry_ref[...] = jnp.broadcast_to(run, carry_ref.shape)
    tri = tri_ref[...]
    for s in range(_NSUB):
        intra = lax.dot_general(
            masked[:, s * _SUB : (s + 1) * _SUB],
            tri,
            (((1,), (0,)), ((), ())),
            preferred_element_type=jnp.float32,
        )
        o_ref[:, s * _SUB : (s + 1) * _SUB] = intra + offs[s]


def _tc_call(x, mask, tri):
    return pl.pallas_call(
        _tc_body,
        grid=(_N // _BT,),
        in_specs=[
            pl.BlockSpec((_RTB, _BT), lambda i: (0, i)),
            pl.BlockSpec((_RTB, _BT), lambda i: (0, i)),
            pl.BlockSpec((_SUB, _SUB), lambda i: (0, 0)),
        ],
        out_specs=pl.BlockSpec((_RTB, _BT), lambda i: (0, i)),
        out_shape=jax.ShapeDtypeStruct((_R, _N), jnp.float32),
        scratch_shapes=[pltpu.VMEM((_RTB, 128), jnp.float32)],
        compiler_params=pltpu.CompilerParams(
            dimension_semantics=("arbitrary",),
        ),
    )(x, mask, tri)


def kernel(x, mask):
    maskf = mask[_ROFF:].astype(jnp.float32)
    tri = jnp.triu(jnp.ones((_SUB, _SUB), jnp.float32))
    sc_out = _sc_call(x, maskf)
    tc_out = _tc_call(x, mask, tri)
    return lax.dynamic_update_slice(tc_out, sc_out, (_ROFF, 0))


# speedup vs baseline: 1.2364x; 1.0325x over previous
import jax
import jax.numpy as jnp
from jax import lax
from jax.experimental import pallas as pl
from jax.experimental.pallas import tpu as pltpu
from jax.experimental.pallas import tpu_sc as plsc

_R, _N = 128, 32768
_RSC = 32
_ROFF = _R - _RSC
_L = 16
_B = _N // 2
_C = _B // _L
_G = _C // _L
_T = _G // _L
_NC, _NS = 2, 16
_NW = _NC * _NS
_RPW = _RSC // _NW
_NB = _RPW * 2
_RT = _ROFF
_RTB = 96
_SUB = 256
_NSUB = 32
_BT = _SUB * _NSUB


def _sc_body(x_hbm, m_hbm, o_hbm, xv, mv, ov, sums, sg, go, off, sems):
    wid = lax.axis_index("s") * _NC + lax.axis_index("c")
    lane = lax.iota(jnp.int32, _L)

    def start_in(b):
        p = b % 2
        row = wid * _RPW + b // 2
        sl = pl.ds((b % 2) * _B, _B)
        return (
            pltpu.async_copy(x_hbm.at[_ROFF + row, sl], xv.at[p], sems.at[p]),
            pltpu.async_copy(m_hbm.at[row, sl], mv.at[p], sems.at[2 + p]),
        )

    def compute_block(p, base):
        pidx = jnp.full((_L,), p, jnp.int32)

        @plsc.parallel_loop(0, _C, unroll=8)
        def _(i):
            o = i * _L
            ov[p, pl.ds(o, _L)] = jnp.cumsum(
                xv[p, pl.ds(o, _L)] * mv[p, pl.ds(o, _L)]
            )

        @plsc.parallel_loop(0, _G, unroll=4)
        def _(g):
            idx = (g * _L + lane) * _L + (_L - 1)
            sums[pl.ds(g * _L, _L)] = plsc.load_gather(ov, [pidx, idx])

        @plsc.parallel_loop(0, _G, unroll=4)
        def _(g):
            sg[pl.ds(g * _L, _L)] = jnp.cumsum(sums[pl.ds(g * _L, _L)])

        def p3(t, carry):
            idx = (t * _L + lane) * _L + (_L - 1)
            gt = plsc.load_gather(sg, [idx])
            st = jnp.cumsum(gt)
            go[pl.ds(t * _L, _L)] = st - gt + carry
            return carry + jnp.sum(gt)
        total = lax.fori_loop(0, _T, p3, base)

        @plsc.parallel_loop(0, _T)
        def _(t):
            gov = go[pl.ds(t * _L, _L)]
            for j in range(_L):
                o = (t * _L + j) * _L
                off[pl.ds(o, _L)] = (
                    sg[pl.ds(o, _L)] - sums[pl.ds(o, _L)] + gov[j]
                )

        @plsc.parallel_loop(0, _G, unroll=4)
        def _(g):
            offv = off[pl.ds(g * _L, _L)]
            for j in range(_L):
                o = (g * _L + j) * _L
                ov[p, pl.ds(o, _L)] = ov[p, pl.ds(o, _L)] + offv[j]

        return total

    in_cps = {0: start_in(0), 1: start_in(1)}
    out_cps = {}
    base = jnp.float32(0.0)
    for b in range(_NB):
        p = b % 2
        if b >= 2:
            out_cps.pop(b - 2).wait()
        for cp in in_cps.pop(b):
            cp.wait()
        if b % 2 == 0:
            base = jnp.float32(0.0)
        base = compute_block(p, base)
        if b + 2 < _NB:
            in_cps[b + 2] = start_in(b + 2)
        row = wid * _RPW + b // 2
        out_cps[b] = pltpu.async_copy(
            ov.at[p], o_hbm.at[row, pl.ds((b % 2) * _B, _B)], sems.at[4 + p]
        )
    for cp in out_cps.values():
        cp.wait()


def _sc_call(x, maskf):
    f = pl.kernel(
        _sc_body,
        out_type=jax.ShapeDtypeStruct((_RSC, _N), jnp.float32),
        mesh=plsc.VectorSubcoreMesh(core_axis_name="c", subcore_axis_name="s"),
        scratch_types=[
            pltpu.VMEM((2, _B), jnp.float32),
            pltpu.VMEM((2, _B), jnp.float32),
            pltpu.VMEM((2, _B), jnp.float32),
            pltpu.VMEM((_C,), jnp.float32),
            pltpu.VMEM((_C,), jnp.float32),
            pltpu.VMEM((_G,), jnp.float32),
            pltpu.VMEM((_C,), jnp.float32),
            pltpu.SemaphoreType.DMA((6,)),
        ],
        compiler_params=pltpu.CompilerParams(needs_layout_passes=False),
    )
    return f(x, maskf)


def _tc_body(x_ref, m_ref, tri_ref, o_ref, carry_ref):
    i = pl.program_id(0)

    @pl.when(i == 0)
    def _():
        carry_ref[...] = jnp.zeros_like(carry_ref)

    masked = jnp.where(m_ref[...], x_ref[...], 0.0)
    run = carry_ref[:, :1]
    offs = []
    for s in range(_NSUB):
        offs.append(run)
        sub_tot = jnp.sum(
            masked[:, s * _SUB : (s + 1) * _SUB], axis=1, keepdims=True
        )
        run = run + sub_tot
    carry_ref[...] = jnp.broadcast_to(run, carry_ref.shape)
    tri = tri_ref[...]
    for s in range(_NSUB):
        intra = lax.dot_general(
            masked[:, s * _SUB : (s + 1) * _SUB],
            tri,
            (((1,), (0,)), ((), ())),
            preferred_element_type=jnp.float32,
        )
        o_ref[:, s * _SUB : (s + 1) * _SUB] = intra + offs[s]


def _tc_call(x, mask, tri):
    return pl.pallas_call(
        _tc_body,
        grid=(_N // _BT,),
        in_specs=[
            pl.BlockSpec((_RTB, _BT), lambda i: (0, i)),
            pl.BlockSpec((_RTB, _BT), lambda i: (0, i)),
            pl.BlockSpec((_SUB, _SUB), lambda i: (0, 0)),
        ],
        out_specs=pl.BlockSpec((_RTB, _BT), lambda i: (0, i)),
        out_shape=jax.ShapeDtypeStruct((_R, _N), jnp.float32),
        scratch_shapes=[pltpu.VMEM((_RTB, 128), jnp.float32)],
        compiler_params=pltpu.CompilerParams(
            dimension_semantics=("arbitrary",),
        ),
    )(x, mask, tri)


def kernel(x, mask):
    maskf = mask[_ROFF:].astype(jnp.float32)
    tri = jnp.triu(jnp.ones((_SUB, _SUB), jnp.float32))
    sc_out = _sc_call(x, maskf)
    tc_out = _tc_call(x, mask, tri)
    return lax.dynamic_update_slice(tc_out, sc_out, (_ROFF, 0))
